# Initial kernel scaffold; baseline (speedup 1.0000x reference)
import jax
import jax.numpy as jnp
from jax.experimental import pallas as pl


def kernel(x, wf1, wd1, wf2, wd2, wlin):
    raise NotImplementedError("write your pallas kernel here")


# speedup vs baseline: 9.5554x; 9.5554x over previous
import functools

import jax
import jax.numpy as jnp
import numpy as np
from jax.experimental import pallas as pl
from jax.experimental.pallas import tpu as pltpu

_EPS = 1e-6
_BN_EPS = 1e-5
_NEG = 0.2

_BB = 2


def _vn_layer(pd, c, mean, istd):
    p = [pd[v * c:(v + 1) * c] for v in range(3)]
    d = [pd[(3 + v) * c:(4 + v) * c] for v in range(3)]
    norm = jnp.sqrt(p[0] * p[0] + p[1] * p[1] + p[2] * p[2]) + _EPS
    scale = (norm - mean) * istd / norm
    p = [pv * scale for pv in p]
    dotp = p[0] * d[0] + p[1] * d[1] + p[2] * d[2]
    dsq = d[0] * d[0] + d[1] * d[1] + d[2] * d[2]
    fac = (1.0 - _NEG) * jnp.where(dotp >= 0, 0.0, dotp / (dsq + _EPS))
    return jnp.concatenate([p[v] - fac * d[v] for v in range(3)], axis=0)


def _norm_moments(p, c):
    norm = jnp.sqrt(p[0:c] * p[0:c] + p[c:2 * c] * p[c:2 * c]
                    + p[2 * c:] * p[2 * c:]) + _EPS
    s = jnp.sum(norm, axis=1, keepdims=True)
    ss = jnp.sum(norm * norm, axis=1, keepdims=True)
    return jnp.concatenate([s, ss], axis=0)


def _pass1_kernel(x_ref, w1f_ref, xbf_ref, mom_ref, *, c1):
    acc = jnp.zeros((2 * c1, 1), jnp.float32)
    for i in range(x_ref.shape[0]):
        xb = x_ref[i].astype(jnp.bfloat16)
        xbf_ref[i] = xb
        p = jnp.dot(w1f_ref[...], xb, preferred_element_type=jnp.float32)
        acc = acc + _norm_moments(p, c1)
    mom_ref[0] = acc


def _pass2_kernel(xbf_ref, w1_ref, m1_ref, s1_ref, w2f_ref, mom_ref, *, c1, c2):
    acc = jnp.zeros((2 * c2, 1), jnp.float32)
    for i in range(xbf_ref.shape[0]):
        pd1 = jnp.dot(w1_ref[...], xbf_ref[i],
                      preferred_element_type=jnp.float32)
        q1 = _vn_layer(pd1, c1, m1_ref[...], s1_ref[...])
        p2 = jnp.dot(w2f_ref[...], q1.astype(jnp.bfloat16),
                     preferred_element_type=jnp.float32)
        acc = acc + _norm_moments(p2, c2)
    mom_ref[0] = acc


def _pass3_kernel(xbf_ref, w1_ref, m1_ref, s1_ref, w2_ref, m2_ref, s2_ref,
                  wl_ref, p_ref, pt_ref, xstd_ref, z_ref, *, c, c1, c2):
    for i in range(xbf_ref.shape[0]):
        xb = xbf_ref[i]
        pd1 = jnp.dot(w1_ref[...], xb, preferred_element_type=jnp.float32)
        q1 = _vn_layer(pd1, c1, m1_ref[...], s1_ref[...])
        pd2 = jnp.dot(w2_ref[...], q1.astype(jnp.bfloat16),
                      preferred_element_type=jnp.float32)
        q2 = _vn_layer(pd2, c2, m2_ref[...], s2_ref[...])
        z = jnp.dot(wl_ref[...], q2.astype(jnp.bfloat16),
                    preferred_element_type=jnp.float32)
        z_ref[i] = z
        xcm = jnp.dot(p_ref[...], xb, preferred_element_type=jnp.float32)
        t = []
        for k in range(3):
            tk = (xcm[0:c] * z[k:k + 1]
                  + xcm[c:2 * c] * z[3 + k:4 + k]
                  + xcm[2 * c:] * z[6 + k:7 + k])
            t.append(tk)
        t = jnp.concatenate(t, axis=0).astype(jnp.bfloat16)
        xstd_ref[i] = jnp.dot(pt_ref[...], t,
                              preferred_element_type=jnp.float32)


def _build_weights(wf1, wd1, wf2, wd2, wlin, c, c1, c2):
    i3 = np.eye(3, dtype=np.float32)
    bf = jnp.bfloat16

    def l1(w):
        return jnp.einsum('vu,cd->vcdu', i3, w).reshape(3 * c1, 3 * c)

    def l2(w):
        return jnp.einsum('vu,cd->vcud', i3, w).reshape(3 * c2, 3 * c1)

    w1f = l1(wf1).astype(bf)
    w1 = jnp.concatenate([l1(wf1), l1(wd1)], axis=0).astype(bf)
    w2f = l2(wf2).astype(bf)
    w2 = jnp.concatenate([l2(wf2), l2(wd2)], axis=0).astype(bf)
    wl = jnp.einsum('vu,kc->vkuc', i3, wlin).reshape(9, 3 * c2).astype(bf)

    perm = np.zeros((3 * c, 3 * c), np.float32)
    for v in range(3):
        for ch in range(c):
            perm[v * c + ch, ch * 3 + v] = 1.0
    p_mat = jnp.asarray(perm, bf)
    pt_mat = jnp.asarray(perm.T, bf)
    return w1f, w1, w2f, w2, wl, p_mat, pt_mat


def kernel(x, wf1, wd1, wf2, wd2, wlin):
    b, c, v, n = x.shape
    assert v == 3
    c1, c2 = wf1.shape[0], wf2.shape[0]
    f32 = jnp.float32
    xr = x.astype(f32).reshape(b, 3 * c, n)

    w1f, w1, w2f, w2, wl, p_mat, pt_mat = _build_weights(
        wf1, wd1, wf2, wd2, wlin, c, c1, c2)

    bb = _BB if b % _BB == 0 else 1
    g = b // bb
    total = float(b * n)

    def xspec(dtype_unused=None):
        return pl.BlockSpec((bb, 3 * c, n), lambda i: (i, 0, 0))

    def wspec(a):
        zeros = (0,) * a.ndim
        return pl.BlockSpec(a.shape, lambda i: zeros)

    def momspec(ch):
        return pl.BlockSpec((1, 2 * ch, 1), lambda i: (i, 0, 0))

    params = pltpu.CompilerParams(dimension_semantics=("parallel",),
                                  vmem_limit_bytes=50 * 1024 * 1024)

    def finalize(mom, ch):
        s = jnp.sum(mom[:, :ch, 0], axis=0)
        ss = jnp.sum(mom[:, ch:, 0], axis=0)
        mean = s / total
        var = ss / total - mean * mean
        istd = jax.lax.rsqrt(var + _BN_EPS)
        return mean.reshape(ch, 1), istd.reshape(ch, 1)

    xbf, mom1 = pl.pallas_call(
        functools.partial(_pass1_kernel, c1=c1),
        grid=(g,),
        in_specs=[xspec(), wspec(w1f)],
        out_specs=(xspec(), momspec(c1)),
        out_shape=(jax.ShapeDtypeStruct((b, 3 * c, n), jnp.bfloat16),
                   jax.ShapeDtypeStruct((g, 2 * c1, 1), f32)),
        compiler_params=params,
    )(xr, w1f)
    mean1, istd1 = finalize(mom1, c1)

    mom2 = pl.pallas_call(
        functools.partial(_pass2_kernel, c1=c1, c2=c2),
        grid=(g,),
        in_specs=[xspec(), wspec(w1), wspec(mean1), wspec(istd1), wspec(w2f)],
        out_specs=momspec(c2),
        out_shape=jax.ShapeDtypeStruct((g, 2 * c2, 1), f32),
        compiler_params=params,
    )(xbf, w1, mean1, istd1, w2f)
    mean2, istd2 = finalize(mom2, c2)

    xstd, z = pl.pallas_call(
        functools.partial(_pass3_kernel, c=c, c1=c1, c2=c2),
        grid=(g,),
        in_specs=[xspec(), wspec(w1), wspec(mean1), wspec(istd1), wspec(w2),
                  wspec(mean2), wspec(istd2), wspec(wl), wspec(p_mat),
                  wspec(pt_mat)],
        out_specs=(xspec(), pl.BlockSpec((bb, 9, n), lambda i: (i, 0, 0))),
        out_shape=(jax.ShapeDtypeStruct((b, 3 * c, n), f32),
                   jax.ShapeDtypeStruct((b, 9, n), f32)),
        compiler_params=params,
    )(xbf, w1, mean1, istd1, w2, mean2, istd2, wl, p_mat, pt_mat)

    return xstd.reshape(b, c, 3, n), z.reshape(b, 3, 3, n)


# speedup vs baseline: 9.7623x; 1.0217x over previous
import functools

import jax
import jax.numpy as jnp
import numpy as np
from jax.experimental import pallas as pl
from jax.experimental.pallas import tpu as pltpu

_EPS = 1e-6
_BN_EPS = 1e-5
_NEG = 0.2

_TN = 4096


def _vn_layer(pd, c, mean, istd):
    p = [pd[v * c:(v + 1) * c] for v in range(3)]
    d = [pd[(3 + v) * c:(4 + v) * c] for v in range(3)]
    norm = jnp.sqrt(p[0] * p[0] + p[1] * p[1] + p[2] * p[2]) + _EPS
    scale = (norm - mean) * istd / norm
    p = [pv * scale for pv in p]
    dotp = p[0] * d[0] + p[1] * d[1] + p[2] * d[2]
    dsq = d[0] * d[0] + d[1] * d[1] + d[2] * d[2]
    fac = (1.0 - _NEG) * jnp.where(dotp >= 0, 0.0, dotp / (dsq + _EPS))
    return jnp.concatenate([p[v] - fac * d[v] for v in range(3)], axis=0)


def _norm_moments(p, c):
    norm = jnp.sqrt(p[0:c] * p[0:c] + p[c:2 * c] * p[c:2 * c]
                    + p[2 * c:] * p[2 * c:]) + _EPS
    s = jnp.sum(norm, axis=1, keepdims=True)
    ss = jnp.sum(norm * norm, axis=1, keepdims=True)
    return jnp.concatenate([s, ss], axis=0)


def _pass1_kernel(x_ref, w1f_ref, xbf_ref, mom_ref, *, c1):
    xb = x_ref[...].astype(jnp.bfloat16)
    xbf_ref[...] = xb
    p = jnp.dot(w1f_ref[...], xb, preferred_element_type=jnp.float32)
    mom_ref[0] = _norm_moments(p, c1)


def _pass2_kernel(xbf_ref, w1_ref, m1_ref, s1_ref, w2f_ref, mom_ref, *, c1, c2):
    pd1 = jnp.dot(w1_ref[...], xbf_ref[...],
                  preferred_element_type=jnp.float32)
    q1 = _vn_layer(pd1, c1, m1_ref[...], s1_ref[...])
    p2 = jnp.dot(w2f_ref[...], q1.astype(jnp.bfloat16),
                 preferred_element_type=jnp.float32)
    mom_ref[0] = _norm_moments(p2, c2)


def _pass3_kernel(xbf_ref, w1_ref, m1_ref, s1_ref, w2_ref, m2_ref, s2_ref,
                  wl_ref, p_ref, pt_ref, xstd_ref, z_ref, *, c, c1, c2):
    xb = xbf_ref[...]
    pd1 = jnp.dot(w1_ref[...], xb, preferred_element_type=jnp.float32)
    q1 = _vn_layer(pd1, c1, m1_ref[...], s1_ref[...])
    pd2 = jnp.dot(w2_ref[...], q1.astype(jnp.bfloat16),
                  preferred_element_type=jnp.float32)
    q2 = _vn_layer(pd2, c2, m2_ref[...], s2_ref[...])
    z = jnp.dot(wl_ref[...], q2.astype(jnp.bfloat16),
                preferred_element_type=jnp.float32)
    z_ref[...] = z
    xcm = jnp.dot(p_ref[...], xb, preferred_element_type=jnp.float32)
    t = []
    for k in range(3):
        tk = (xcm[0:c] * z[k:k + 1]
              + xcm[c:2 * c] * z[3 + k:4 + k]
              + xcm[2 * c:] * z[6 + k:7 + k])
        t.append(tk)
    t = jnp.concatenate(t, axis=0).astype(jnp.bfloat16)
    xstd_ref[...] = jnp.dot(pt_ref[...], t,
                            preferred_element_type=jnp.float32)


def _build_weights(wf1, wd1, wf2, wd2, wlin, c, c1, c2):
    i3 = np.eye(3, dtype=np.float32)
    bf = jnp.bfloat16

    def l1(w):
        return jnp.einsum('vu,cd->vcdu', i3, w).reshape(3 * c1, 3 * c)

    def l2(w):
        return jnp.einsum('vu,cd->vcud', i3, w).reshape(3 * c2, 3 * c1)

    w1f = l1(wf1).astype(bf)
    w1 = jnp.concatenate([l1(wf1), l1(wd1)], axis=0).astype(bf)
    w2f = l2(wf2).astype(bf)
    w2 = jnp.concatenate([l2(wf2), l2(wd2)], axis=0).astype(bf)
    wl = jnp.einsum('vu,kc->vkuc', i3, wlin).reshape(9, 3 * c2).astype(bf)

    perm = np.zeros((3 * c, 3 * c), np.float32)
    for v in range(3):
        for ch in range(c):
            perm[v * c + ch, ch * 3 + v] = 1.0
    p_mat = jnp.asarray(perm, bf)
    pt_mat = jnp.asarray(perm.T, bf)
    return w1f, w1, w2f, w2, wl, p_mat, pt_mat


def kernel(x, wf1, wd1, wf2, wd2, wlin):
    b, c, v, n = x.shape
    assert v == 3
    c1, c2 = wf1.shape[0], wf2.shape[0]
    f32 = jnp.float32
    m = b * n
    xt = jnp.transpose(x.astype(f32), (1, 2, 0, 3)).reshape(3 * c, m)

    w1f, w1, w2f, w2, wl, p_mat, pt_mat = _build_weights(
        wf1, wd1, wf2, wd2, wlin, c, c1, c2)

    tn = _TN if m % _TN == 0 else n
    g = m // tn
    total = float(m)

    xspec = pl.BlockSpec((3 * c, tn), lambda i: (0, i))

    def wspec(a):
        zeros = (0,) * a.ndim
        return pl.BlockSpec(a.shape, lambda i: zeros)

    def momspec(ch):
        return pl.BlockSpec((1, 2 * ch, 1), lambda i: (i, 0, 0))

    params = pltpu.CompilerParams(dimension_semantics=("parallel",),
                                  vmem_limit_bytes=50 * 1024 * 1024)

    def finalize(mom, ch):
        s = jnp.sum(mom[:, :ch, 0], axis=0)
        ss = jnp.sum(mom[:, ch:, 0], axis=0)
        mean = s / total
        var = ss / total - mean * mean
        istd = jax.lax.rsqrt(var + _BN_EPS)
        return mean.reshape(ch, 1), istd.reshape(ch, 1)

    xbf, mom1 = pl.pallas_call(
        functools.partial(_pass1_kernel, c1=c1),
        grid=(g,),
        in_specs=[xspec, wspec(w1f)],
        out_specs=(xspec, momspec(c1)),
        out_shape=(jax.ShapeDtypeStruct((3 * c, m), jnp.bfloat16),
                   jax.ShapeDtypeStruct((g, 2 * c1, 1), f32)),
        compiler_params=params,
    )(xt, w1f)
    mean1, istd1 = finalize(mom1, c1)

    mom2 = pl.pallas_call(
        functools.partial(_pass2_kernel, c1=c1, c2=c2),
        grid=(g,),
        in_specs=[xspec, wspec(w1), wspec(mean1), wspec(istd1), wspec(w2f)],
        out_specs=momspec(c2),
        out_shape=jax.ShapeDtypeStruct((g, 2 * c2, 1), f32),
        compiler_params=params,
    )(xbf, w1, mean1, istd1, w2f)
    mean2, istd2 = finalize(mom2, c2)

    xstd_t, z_t = pl.pallas_call(
        functools.partial(_pass3_kernel, c=c, c1=c1, c2=c2),
        grid=(g,),
        in_specs=[xspec, wspec(w1), wspec(mean1), wspec(istd1), wspec(w2),
                  wspec(mean2), wspec(istd2), wspec(wl), wspec(p_mat),
                  wspec(pt_mat)],
        out_specs=(xspec, pl.BlockSpec((9, tn), lambda i: (0, i))),
        out_shape=(jax.ShapeDtypeStruct((3 * c, m), f32),
                   jax.ShapeDtypeStruct((9, m), f32)),
        compiler_params=params,
    )(xbf, w1, mean1, istd1, w2, mean2, istd2, wl, p_mat, pt_mat)

    x_std = jnp.transpose(xstd_t.reshape(c, 3, b, n), (2, 0, 1, 3))
    z0 = jnp.transpose(z_t.reshape(3, 3, b, n), (2, 0, 1, 3))
    return x_std, z0


# speedup vs baseline: 26.0914x; 2.6727x over previous
import functools

import jax
import jax.numpy as jnp
import numpy as np
from jax.experimental import pallas as pl
from jax.experimental.pallas import tpu as pltpu

_EPS = 1e-6
_BN_EPS = 1e-5
_NEG = 0.2

_BB = 2


def _vn_layer(pd, c, mean, istd):
    p = [pd[v * c:(v + 1) * c] for v in range(3)]
    d = [pd[(3 + v) * c:(4 + v) * c] for v in range(3)]
    norm = jnp.sqrt(p[0] * p[0] + p[1] * p[1] + p[2] * p[2]) + _EPS
    scale = (norm - mean) * istd / norm
    p = [pv * scale for pv in p]
    dotp = p[0] * d[0] + p[1] * d[1] + p[2] * d[2]
    dsq = d[0] * d[0] + d[1] * d[1] + d[2] * d[2]
    fac = (1.0 - _NEG) * jnp.where(dotp >= 0, 0.0, dotp / (dsq + _EPS))
    return jnp.concatenate([p[v] - fac * d[v] for v in range(3)], axis=0)


def _norm_moments(p, c):
    norm = jnp.sqrt(p[0:c] * p[0:c] + p[c:2 * c] * p[c:2 * c]
                    + p[2 * c:] * p[2 * c:]) + _EPS
    s = jnp.sum(norm, axis=1, keepdims=True)
    ss = jnp.sum(norm * norm, axis=1, keepdims=True)
    return jnp.concatenate([s, ss], axis=0)


def _pass1_kernel(x_ref, w1f_ref, xbf_ref, mom_ref, *, c1):
    acc = jnp.zeros((2 * c1, 1), jnp.float32)
    for i in range(x_ref.shape[0]):
        xb = x_ref[i].astype(jnp.bfloat16)
        xbf_ref[i] = xb
        p = jnp.dot(w1f_ref[...], xb, preferred_element_type=jnp.float32)
        acc = acc + _norm_moments(p, c1)
    mom_ref[0] = acc


def _pass2_kernel(xbf_ref, w1_ref, m1_ref, s1_ref, w2f_ref, mom_ref, *, c1, c2):
    acc = jnp.zeros((2 * c2, 1), jnp.float32)
    for i in range(xbf_ref.shape[0]):
        pd1 = jnp.dot(w1_ref[...], xbf_ref[i],
                      preferred_element_type=jnp.float32)
        q1 = _vn_layer(pd1, c1, m1_ref[...], s1_ref[...])
        p2 = jnp.dot(w2f_ref[...], q1.astype(jnp.bfloat16),
                     preferred_element_type=jnp.float32)
        acc = acc + _norm_moments(p2, c2)
    mom_ref[0] = acc


def _pass3_kernel(xbf_ref, w1_ref, m1_ref, s1_ref, w2_ref, m2_ref, s2_ref,
                  wl_ref, xstd_ref, z_ref, *, c, c1, c2):
    for i in range(xbf_ref.shape[0]):
        xb = xbf_ref[i]
        pd1 = jnp.dot(w1_ref[...], xb, preferred_element_type=jnp.float32)
        q1 = _vn_layer(pd1, c1, m1_ref[...], s1_ref[...])
        pd2 = jnp.dot(w2_ref[...], q1.astype(jnp.bfloat16),
                      preferred_element_type=jnp.float32)
        q2 = _vn_layer(pd2, c2, m2_ref[...], s2_ref[...])
        z = jnp.dot(wl_ref[...], q2.astype(jnp.bfloat16),
                    preferred_element_type=jnp.float32)
        z_ref[i] = z
        t = []
        for k in range(3):
            tk = (xb[0:c].astype(jnp.float32) * z[k:k + 1]
                  + xb[c:2 * c].astype(jnp.float32) * z[3 + k:4 + k]
                  + xb[2 * c:].astype(jnp.float32) * z[6 + k:7 + k])
            t.append(tk)
        xstd_ref[i] = jnp.concatenate(t, axis=0)


def _build_weights(wf1, wd1, wf2, wd2, wlin, c, c1, c2):
    i3 = np.eye(3, dtype=np.float32)
    bf = jnp.bfloat16

    def blk(w):
        co, ci = w.shape
        return jnp.einsum('vu,cd->vcud', i3, w).reshape(3 * co, 3 * ci)

    w1f = blk(wf1).astype(bf)
    w1 = jnp.concatenate([blk(wf1), blk(wd1)], axis=0).astype(bf)
    w2f = blk(wf2).astype(bf)
    w2 = jnp.concatenate([blk(wf2), blk(wd2)], axis=0).astype(bf)
    wl = jnp.einsum('vu,kc->vkuc', i3, wlin).reshape(9, 3 * c2).astype(bf)
    return w1f, w1, w2f, w2, wl


def kernel(x, wf1, wd1, wf2, wd2, wlin):
    b, c, v, n = x.shape
    assert v == 3
    c1, c2 = wf1.shape[0], wf2.shape[0]
    f32 = jnp.float32
    xm = jnp.transpose(x.astype(f32), (0, 2, 1, 3)).reshape(b, 3 * c, n)

    w1f, w1, w2f, w2, wl = _build_weights(wf1, wd1, wf2, wd2, wlin, c, c1, c2)

    bb = _BB if b % _BB == 0 else 1
    g = b // bb
    total = float(b * n)

    xspec = pl.BlockSpec((bb, 3 * c, n), lambda i: (i, 0, 0))

    def wspec(a):
        zeros = (0,) * a.ndim
        return pl.BlockSpec(a.shape, lambda i: zeros)

    def momspec(ch):
        return pl.BlockSpec((1, 2 * ch, 1), lambda i: (i, 0, 0))

    params = pltpu.CompilerParams(dimension_semantics=("parallel",),
                                  vmem_limit_bytes=50 * 1024 * 1024)

    def finalize(mom, ch):
        s = jnp.sum(mom[:, :ch, 0], axis=0)
        ss = jnp.sum(mom[:, ch:, 0], axis=0)
        mean = s / total
        var = ss / total - mean * mean
        istd = jax.lax.rsqrt(var + _BN_EPS)
        return mean.reshape(ch, 1), istd.reshape(ch, 1)

    xbf, mom1 = pl.pallas_call(
        functools.partial(_pass1_kernel, c1=c1),
        grid=(g,),
        in_specs=[xspec, wspec(w1f)],
        out_specs=(xspec, momspec(c1)),
        out_shape=(jax.ShapeDtypeStruct((b, 3 * c, n), jnp.bfloat16),
                   jax.ShapeDtypeStruct((g, 2 * c1, 1), f32)),
        compiler_params=params,
    )(xm, w1f)
    mean1, istd1 = finalize(mom1, c1)

    mom2 = pl.pallas_call(
        functools.partial(_pass2_kernel, c1=c1, c2=c2),
        grid=(g,),
        in_specs=[xspec, wspec(w1), wspec(mean1), wspec(istd1), wspec(w2f)],
        out_specs=momspec(c2),
        out_shape=jax.ShapeDtypeStruct((g, 2 * c2, 1), f32),
        compiler_params=params,
    )(xbf, w1, mean1, istd1, w2f)
    mean2, istd2 = finalize(mom2, c2)

    xstd_m, z_m = pl.pallas_call(
        functools.partial(_pass3_kernel, c=c, c1=c1, c2=c2),
        grid=(g,),
        in_specs=[xspec, wspec(w1), wspec(mean1), wspec(istd1), wspec(w2),
                  wspec(mean2), wspec(istd2), wspec(wl)],
        out_specs=(xspec, pl.BlockSpec((bb, 9, n), lambda i: (i, 0, 0))),
        out_shape=(jax.ShapeDtypeStruct((b, 3 * c, n), f32),
                   jax.ShapeDtypeStruct((b, 9, n), f32)),
        compiler_params=params,
    )(xbf, w1, mean1, istd1, w2, mean2, istd2, wl)

    x_std = jnp.transpose(xstd_m.reshape(b, 3, c, n), (0, 2, 1, 3))
    z0 = z_m.reshape(b, 3, 3, n)
    return x_std, z0
